# baseline (speedup 1.0000x reference)
import jax
import jax.numpy as jnp
from jax import lax
from jax.experimental import pallas as pl
from jax.experimental.pallas import tpu as pltpu
from jax.experimental.pallas import tpu_sc as plsc

N = 10000
D = 128
E = 320000
NW = 32
NPW = 320
NP = NW * NPW
C = 8000
NCHUNK = E // C
B = 128
GPB = B // 16

NEG_INF = float("-inf")


def _mm_body(x_ref, wn_ref, u_ref, xw_ref, xwu_ref):
    xw = jnp.dot(x_ref[...], wn_ref[...], preferred_element_type=jnp.float32)
    xw_ref[...] = xw
    xwu_ref[...] = jnp.dot(xw, u_ref[...], preferred_element_type=jnp.float32)


def _matmuls(xp, Wn, U):
    blk = 1024
    return pl.pallas_call(
        _mm_body,
        grid=(NP // blk,),
        in_specs=[
            pl.BlockSpec((blk, D), lambda i: (i, 0)),
            pl.BlockSpec((D, D), lambda i: (0, 0)),
            pl.BlockSpec((D, D), lambda i: (0, 0)),
        ],
        out_specs=[
            pl.BlockSpec((blk, D), lambda i: (i, 0)),
            pl.BlockSpec((blk, D), lambda i: (i, 0)),
        ],
        out_shape=[
            jax.ShapeDtypeStruct((NP, D), jnp.float32),
            jax.ShapeDtypeStruct((NP, D), jnp.float32),
        ],
    )(xp, Wn, U)


def _edge_body(xwu_hbm, xw_hbm, src_hbm, dst_hbm, out_hbm,
               acc, src_chunk, dst_chunk, sel_src, sel_dst,
               srows, drows, probe, sem1, sem2):
    cid = lax.axis_index("c")
    sid = lax.axis_index("s")
    wid = sid * 2 + cid
    lo = (wid * NPW).astype(jnp.int32)
    hi = lo + NPW
    iota = lax.broadcasted_iota(jnp.int32, (16,), 0)

    def init_acc(r, _):
        for c in range(D // 16):
            acc[r, pl.ds(c * 16, 16)] = jnp.full((16,), NEG_INF, jnp.float32)
        return 0
    lax.fori_loop(0, NPW, init_acc, 0)

    def init_sel(i, _):
        sel_src[pl.ds(i * 16, 16)] = jnp.zeros((16,), jnp.int32)
        sel_dst[pl.ds(i * 16, 16)] = jnp.full((16,), lo, jnp.int32)
        return 0
    lax.fori_loop(0, C // 16, init_sel, 0)

    def process_batch(boff, cnt):
        cps = []
        for q in range(4):
            cps.append(pltpu.async_copy(
                xwu_hbm.at[sel_src.at[pl.ds(boff + q * 32, 32)]],
                srows.at[pl.ds(q * 32, 32)], sem1))
            cps.append(pltpu.async_copy(
                xwu_hbm.at[sel_dst.at[pl.ds(boff + q * 32, 32)]],
                drows.at[pl.ds(q * 32, 32)], sem2))
        for cp in cps:
            cp.wait()

        def group(g, _):
            base = boff + g * 16
            dv = sel_dst[pl.ds(base, 16)]
            dl = dv - lo
            valid = (base + iota) < cnt
            e16 = g * 16 + iota

            def dot_step(db, acc_dot):
                res = acc_dot
                for j in range(8):
                    col = jnp.full((16,), db * 8 + j, jnp.int32)
                    s = plsc.load_gather(srows, [e16, col])
                    t = plsc.load_gather(drows, [e16, col])
                    res = res + s * t
                return res
            dot = lax.fori_loop(0, D // 8, dot_step, jnp.zeros((16,), jnp.float32))
            gate = 1.0 / (1.0 + jnp.exp(-dot))

            def has_pending(pending):
                return jnp.max(plsc.all_reduce_population_count(pending)) > 0

            def round_body(pending):
                plsc.store_scatter(probe, [dl], iota, mask=pending)
                back = plsc.load_gather(probe, [dl], mask=pending)
                winners = pending & (back == iota)

                def upd_step(db, _):
                    for j in range(8):
                        col = jnp.full((16,), db * 8 + j, jnp.int32)
                        s = plsc.load_gather(srows, [e16, col])
                        m = s * gate
                        a = plsc.load_gather(acc, [dl, col], mask=winners)
                        plsc.store_scatter(acc, [dl, col], jnp.maximum(a, m),
                                           mask=winners)
                    return 0
                lax.fori_loop(0, D // 8, upd_step, 0)
                return pending & jnp.logical_not(winners)

            lax.while_loop(has_pending, round_body, valid)
            return 0

        pass

    def chunk_body(k, _):
        pltpu.sync_copy(src_hbm.at[pl.ds(k * C, C)], src_chunk)
        pltpu.sync_copy(dst_hbm.at[pl.ds(k * C, C)], dst_chunk)

        def filt(i, cntv):
            cv = cntv
            for u in range(4):
                off = (i * 4 + u) * 16
                d16 = dst_chunk[pl.ds(off, 16)]
                s16 = src_chunk[pl.ds(off, 16)]
                m = (d16 >= lo) & (d16 < hi)
                pos = cv + plsc.cumsum(m.astype(jnp.int32)) - 1
                plsc.store_scatter(sel_dst, [pos], d16, mask=m)
                plsc.store_scatter(sel_src, [pos], s16, mask=m)
                cv = cv + plsc.all_reduce_population_count(m)
            return cv
        cntv = lax.fori_loop(0, C // 64, filt, jnp.zeros((16,), jnp.int32))
        cnt = jnp.max(cntv)

        nb = (cnt + (B - 1)) >> 7

        def batch(b, _):
            process_batch(b * B, cnt)
            return 0
        lax.fori_loop(0, nb, batch, 0)
        return 0

    lax.fori_loop(0, NCHUNK, chunk_body, 0)

    WB = 64

    def wb(blk, _):
        pltpu.sync_copy(xw_hbm.at[pl.ds(lo + blk * WB, WB)], srows.at[pl.ds(0, WB)])

        def row(r, _):
            for c in range(D // 16):
                a = acc[blk * WB + r, pl.ds(c * 16, 16)]
                az = jnp.where(a == NEG_INF, 0.0, a)
                s = srows[r, pl.ds(c * 16, 16)] + az
                drows[r, pl.ds(c * 16, 16)] = jnp.maximum(s, 0.01 * s)
            return 0
        lax.fori_loop(0, WB, row, 0)
        pltpu.sync_copy(drows.at[pl.ds(0, WB)], out_hbm.at[pl.ds(lo + blk * WB, WB)])
        return 0
    lax.fori_loop(0, NPW // WB, wb, 0)


def _edge_layer(xwu, xw, src, dst):
    mesh = plsc.VectorSubcoreMesh(core_axis_name="c", subcore_axis_name="s")
    f = pl.kernel(
        _edge_body,
        out_type=jax.ShapeDtypeStruct((NP, D), jnp.float32),
        mesh=mesh,
        compiler_params=pltpu.CompilerParams(needs_layout_passes=False),
        scratch_types=[
            pltpu.VMEM((NPW, D), jnp.float32),
            pltpu.VMEM((C,), jnp.int32),
            pltpu.VMEM((C,), jnp.int32),
            pltpu.VMEM((C,), jnp.int32),
            pltpu.VMEM((C,), jnp.int32),
            pltpu.VMEM((B, D), jnp.float32),
            pltpu.VMEM((B, D), jnp.float32),
            pltpu.VMEM((NPW,), jnp.int32),
            pltpu.SemaphoreType.DMA,
            pltpu.SemaphoreType.DMA,
        ],
    )
    return f(xwu, xw, src, dst)


def kernel(x, edge_index, edge_attr, Wn1, We1, U1, Wn2, We2, U2):
    ei = edge_index.astype(jnp.int32)
    src = ei[0]
    dst = ei[1]
    xp = jnp.pad(x, ((0, NP - N), (0, 0)))
    xw1, xwu1 = _matmuls(xp, Wn1, U1)
    c1 = _edge_layer(xwu1, xw1, src, dst)
    xw2, xwu2 = _matmuls(c1, Wn2, U2)
    c2 = _edge_layer(xwu2, xw2, src, dst)
    return c2[:N]


# speedup vs baseline: 1.4446x; 1.4446x over previous
import jax
import jax.numpy as jnp
from jax import lax
from jax.experimental import pallas as pl
from jax.experimental.pallas import tpu as pltpu
from jax.experimental.pallas import tpu_sc as plsc

N = 10000
D = 128
E = 320000
NW = 32
NPW = 320
NP = NW * NPW
C = 6400
NCHUNK = E // C
B = 128
GPB = B // 16

NEG_INF = float("-inf")


def _mm_body(x_ref, wn_ref, u_ref, xw_ref, xwu_ref):
    xw = jnp.dot(x_ref[...], wn_ref[...], preferred_element_type=jnp.float32)
    xw_ref[...] = xw
    xwu_ref[...] = jnp.dot(xw, u_ref[...], preferred_element_type=jnp.float32)


def _matmuls(xp, Wn, U):
    blk = 1024
    return pl.pallas_call(
        _mm_body,
        grid=(NP // blk,),
        in_specs=[
            pl.BlockSpec((blk, D), lambda i: (i, 0)),
            pl.BlockSpec((D, D), lambda i: (0, 0)),
            pl.BlockSpec((D, D), lambda i: (0, 0)),
        ],
        out_specs=[
            pl.BlockSpec((blk, D), lambda i: (i, 0)),
            pl.BlockSpec((blk, D), lambda i: (i, 0)),
        ],
        out_shape=[
            jax.ShapeDtypeStruct((NP, D), jnp.float32),
            jax.ShapeDtypeStruct((NP, D), jnp.float32),
        ],
    )(xp, Wn, U)


def _edge_body(xwu_hbm, xw_hbm, src_hbm, dst_hbm, out_hbm,
               acc, src_chunk, dst_chunk, sel_src, sel_dst,
               srows, dstloc, probe, sem1):
    cid = lax.axis_index("c")
    sid = lax.axis_index("s")
    wid = sid * 2 + cid
    lo = (wid * NPW).astype(jnp.int32)
    hi = lo + NPW
    iota = lax.broadcasted_iota(jnp.int32, (16,), 0)

    pltpu.sync_copy(xwu_hbm.at[pl.ds(lo, NPW)], dstloc)

    def init_acc(r, _):
        for c in range(D // 16):
            acc[r, pl.ds(c * 16, 16)] = jnp.full((16,), NEG_INF, jnp.float32)
        return 0
    lax.fori_loop(0, NPW, init_acc, 0)

    def init_sel(i, _):
        sel_src[pl.ds(i * 16, 16)] = jnp.zeros((16,), jnp.int32)
        sel_dst[pl.ds(i * 16, 16)] = jnp.full((16,), lo, jnp.int32)
        return 0
    lax.fori_loop(0, C // 16, init_sel, 0)

    def process_batch(boff, cnt):
        cp1 = pltpu.async_copy(xwu_hbm.at[sel_src.at[pl.ds(boff, B)]], srows, sem1)
        cp1.wait()

        def group(g, _):
            base = boff + g * 16
            dv = sel_dst[pl.ds(base, 16)]
            dl = dv - lo
            valid = (base + iota) < cnt
            e16 = g * 16 + iota

            def dot_step(db, acc_dot):
                res = acc_dot
                for j in range(8):
                    col = (jnp.full((16,), db * 8 + j, jnp.int32) + iota) & 127
                    s = plsc.load_gather(srows, [e16, col])
                    t = plsc.load_gather(dstloc, [dl, col])
                    res = res + s * t
                return res
            dot = lax.fori_loop(0, D // 8, dot_step, jnp.zeros((16,), jnp.float32))
            gate = 1.0 / (1.0 + jnp.exp(-dot))

            def has_pending(pending):
                return jnp.max(plsc.all_reduce_population_count(pending)) > 0

            def round_body(pending):
                plsc.store_scatter(probe, [dl], iota, mask=pending)
                back = plsc.load_gather(probe, [dl], mask=pending)
                winners = pending & (back == iota)

                def upd_step(db, _):
                    for j in range(8):
                        col = (jnp.full((16,), db * 8 + j, jnp.int32) + iota) & 127
                        s = plsc.load_gather(srows, [e16, col])
                        m = s * gate
                        a = plsc.load_gather(acc, [dl, col], mask=winners)
                        plsc.store_scatter(acc, [dl, col], jnp.maximum(a, m),
                                           mask=winners)
                    return 0
                lax.fori_loop(0, D // 8, upd_step, 0)
                return pending & jnp.logical_not(winners)

            lax.while_loop(has_pending, round_body, valid)
            return 0

        lax.fori_loop(0, GPB, group, 0)

    def chunk_body(k, _):
        pltpu.sync_copy(src_hbm.at[pl.ds(k * C, C)], src_chunk)
        pltpu.sync_copy(dst_hbm.at[pl.ds(k * C, C)], dst_chunk)

        def filt(i, cntv):
            cv = cntv
            for u in range(4):
                off = (i * 4 + u) * 16
                d16 = dst_chunk[pl.ds(off, 16)]
                s16 = src_chunk[pl.ds(off, 16)]
                m = (d16 >= lo) & (d16 < hi)
                pos = cv + plsc.cumsum(m.astype(jnp.int32)) - 1
                plsc.store_scatter(sel_dst, [pos], d16, mask=m)
                plsc.store_scatter(sel_src, [pos], s16, mask=m)
                cv = cv + plsc.all_reduce_population_count(m)
            return cv
        cntv = lax.fori_loop(0, C // 64, filt, jnp.zeros((16,), jnp.int32))
        cnt = jnp.max(cntv)

        nb = (cnt + (B - 1)) >> 7

        def batch(b, _):
            process_batch(b * B, cnt)
            return 0
        lax.fori_loop(0, nb, batch, 0)
        return 0

    lax.fori_loop(0, NCHUNK, chunk_body, 0)

    WB = 64

    def wb(blk, _):
        pltpu.sync_copy(xw_hbm.at[pl.ds(lo + blk * WB, WB)],
                        srows.at[pl.ds(0, WB)])

        def row(r, _):
            for c in range(D // 16):
                a = acc[blk * WB + r, pl.ds(c * 16, 16)]
                az = jnp.where(a == NEG_INF, 0.0, a)
                s = srows[r, pl.ds(c * 16, 16)] + az
                srows[r, pl.ds(c * 16, 16)] = jnp.maximum(s, 0.01 * s)
            return 0
        lax.fori_loop(0, WB, row, 0)
        pltpu.sync_copy(srows.at[pl.ds(0, WB)],
                        out_hbm.at[pl.ds(lo + blk * WB, WB)])
        return 0
    lax.fori_loop(0, NPW // WB, wb, 0)


def _edge_layer(xwu, xw, src, dst):
    mesh = plsc.VectorSubcoreMesh(core_axis_name="c", subcore_axis_name="s")
    f = pl.kernel(
        _edge_body,
        out_type=jax.ShapeDtypeStruct((NP, D), jnp.float32),
        mesh=mesh,
        compiler_params=pltpu.CompilerParams(needs_layout_passes=False, internal_scratch_in_bytes=0),
        scratch_types=[
            pltpu.VMEM((NPW, D), jnp.float32),
            pltpu.VMEM((C,), jnp.int32),
            pltpu.VMEM((C,), jnp.int32),
            pltpu.VMEM((C,), jnp.int32),
            pltpu.VMEM((C,), jnp.int32),
            pltpu.VMEM((B, D), jnp.float32),
            pltpu.VMEM((NPW, D), jnp.float32),
            pltpu.VMEM((NPW,), jnp.int32),
            pltpu.SemaphoreType.DMA,
        ],
    )
    return f(xwu, xw, src, dst)


def kernel(x, edge_index, edge_attr, Wn1, We1, U1, Wn2, We2, U2):
    ei = edge_index.astype(jnp.int32)
    src = ei[0]
    dst = ei[1]
    xp = jnp.pad(x, ((0, NP - N), (0, 0)))
    xw1, xwu1 = _matmuls(xp, Wn1, U1)
    c1 = _edge_layer(xwu1, xw1, src, dst)
    xw2, xwu2 = _matmuls(c1, Wn2, U2)
    c2 = _edge_layer(xwu2, xw2, src, dst)
    return c2[:N]


# speedup vs baseline: 2.2561x; 1.5617x over previous
import jax
import jax.numpy as jnp
from jax import lax
from jax.experimental import pallas as pl
from jax.experimental.pallas import tpu as pltpu
from jax.experimental.pallas import tpu_sc as plsc

N = 10000
D = 128
E = 320000
NW = 32
NPW = 320
NP = NW * NPW
C = 6400
NCHUNK = E // C
B = 64
GPB = B // 16

NEG_INF = float("-inf")


def _mm_body(x_ref, wn_ref, u_ref, xw_ref, xwu_ref):
    xw = jnp.dot(x_ref[...], wn_ref[...], preferred_element_type=jnp.float32)
    xw_ref[...] = xw
    xwu_ref[...] = jnp.dot(xw, u_ref[...], preferred_element_type=jnp.float32)


def _matmuls(xp, Wn, U):
    blk = 1024
    return pl.pallas_call(
        _mm_body,
        grid=(NP // blk,),
        in_specs=[
            pl.BlockSpec((blk, D), lambda i: (i, 0)),
            pl.BlockSpec((D, D), lambda i: (0, 0)),
            pl.BlockSpec((D, D), lambda i: (0, 0)),
        ],
        out_specs=[
            pl.BlockSpec((blk, D), lambda i: (i, 0)),
            pl.BlockSpec((blk, D), lambda i: (i, 0)),
        ],
        out_shape=[
            jax.ShapeDtypeStruct((NP, D), jnp.float32),
            jax.ShapeDtypeStruct((NP, D), jnp.float32),
        ],
    )(xp, Wn, U)


def _edge_body(xwu_hbm, xw_hbm, src_hbm, dst_hbm, out_hbm,
               acc, src_chunk, dst_chunk, sel_src, sel_dst,
               srows3, dstloc, probe, sem0, sem1):
    cid = lax.axis_index("c")
    sid = lax.axis_index("s")
    wid = sid * 2 + cid
    lo = (wid * NPW).astype(jnp.int32)
    hi = lo + NPW
    iota = lax.broadcasted_iota(jnp.int32, (16,), 0)

    pltpu.sync_copy(xwu_hbm.at[pl.ds(lo, NPW)], dstloc)

    def init_acc(r, _):
        for c in range(D // 16):
            acc[r, pl.ds(c * 16, 16)] = jnp.full((16,), NEG_INF, jnp.float32)
        return 0
    lax.fori_loop(0, NPW, init_acc, 0)

    def init_sel(i, _):
        sel_src[pl.ds(i * 16, 16)] = jnp.zeros((16,), jnp.int32)
        sel_dst[pl.ds(i * 16, 16)] = jnp.full((16,), lo, jnp.int32)
        return 0
    lax.fori_loop(0, (C + B) // 16, init_sel, 0)

    def issue(boff, p):

        def go(sem):
            pltpu.async_copy(xwu_hbm.at[sel_src.at[pl.ds(boff, B)]],
                             srows3.at[p], sem)

        @pl.when(p == 0)
        def _():
            go(sem0)

        @pl.when(p == 1)
        def _():
            go(sem1)

    def wait(p):

        def drain(sem):
            pltpu.make_async_copy(xwu_hbm.at[pl.ds(0, B)], srows3.at[p],
                                  sem).wait()

        @pl.when(p == 0)
        def _():
            drain(sem0)

        @pl.when(p == 1)
        def _():
            drain(sem1)

    def compute(p, soff, cnt):
        srows = srows3.at[p]

        def group(g, _):
            sbase = soff + g * 16
            dv = sel_dst[pl.ds(sbase, 16)]
            dl = dv - lo
            valid = (sbase + iota) < cnt
            e16 = g * 16 + iota

            def dot_step(db, acc_dot):
                res = acc_dot
                for j in range(8):
                    col = (jnp.full((16,), db * 8 + j, jnp.int32) + iota) & 127
                    s = plsc.load_gather(srows, [e16, col])
                    t = plsc.load_gather(dstloc, [dl, col])
                    res = res + s * t
                return res
            dot = lax.fori_loop(0, D // 8, dot_step, jnp.zeros((16,), jnp.float32))
            gate = 1.0 / (1.0 + jnp.exp(-dot))

            def has_pending(pending):
                return jnp.max(plsc.all_reduce_population_count(pending)) > 0

            def round_body(pending):
                plsc.store_scatter(probe, [dl], iota, mask=pending)
                back = plsc.load_gather(probe, [dl], mask=pending)
                winners = pending & (back == iota)

                def upd_step(db, _):
                    for j in range(8):
                        col = (jnp.full((16,), db * 8 + j, jnp.int32) + iota) & 127
                        s = plsc.load_gather(srows, [e16, col])
                        m = s * gate
                        a = plsc.load_gather(acc, [dl, col], mask=winners)
                        plsc.store_scatter(acc, [dl, col], jnp.maximum(a, m),
                                           mask=winners)
                    return 0
                lax.fori_loop(0, D // 8, upd_step, 0)
                return pending & jnp.logical_not(winners)

            lax.while_loop(has_pending, round_body, valid)
            return 0

        lax.fori_loop(0, GPB, group, 0)

    def chunk_body(k, rem):
        pltpu.sync_copy(src_hbm.at[pl.ds(k * C, C)], src_chunk)
        pltpu.sync_copy(dst_hbm.at[pl.ds(k * C, C)], dst_chunk)

        def filt(i, cntv):
            cv = cntv
            for u in range(4):
                off = (i * 4 + u) * 16
                d16 = dst_chunk[pl.ds(off, 16)]
                s16 = src_chunk[pl.ds(off, 16)]
                m = (d16 >= lo) & (d16 < hi)
                pos = cv + plsc.cumsum(m.astype(jnp.int32)) - 1
                plsc.store_scatter(sel_dst, [pos], d16, mask=m)
                plsc.store_scatter(sel_src, [pos], s16, mask=m)
                cv = cv + plsc.all_reduce_population_count(m)
            return cv
        cntv = lax.fori_loop(0, C // 64, filt,
                             jnp.full((16,), rem, jnp.int32))
        cnt = jnp.max(cntv)

        nb = cnt >> 6

        @pl.when(nb > 0)
        def _():
            issue(0, jnp.int32(0))

        def batch(b, _):
            p = b & 1

            @pl.when(b + 1 < nb)
            def _():
                issue((b + 1) * B, 1 - p)

            wait(p)
            compute(p, b * B, cnt)
            return 0
        lax.fori_loop(0, nb, batch, 0)

        base = nb << 6
        for i in range(B // 16):
            vs = sel_src[pl.ds(base + i * 16, 16)]
            vd = sel_dst[pl.ds(base + i * 16, 16)]
            sel_src[pl.ds(i * 16, 16)] = vs
            sel_dst[pl.ds(i * 16, 16)] = vd
        return cnt - base

    rem = lax.fori_loop(0, NCHUNK, chunk_body, jnp.int32(0))

    @pl.when(rem > 0)
    def _():
        issue(0, jnp.int32(0))
        wait(jnp.int32(0))
        compute(jnp.int32(0), 0, rem)

    WB = 64

    def wb(blk, _):
        wbuf = srows3.at[0]
        pltpu.sync_copy(xw_hbm.at[pl.ds(lo + blk * WB, WB)], wbuf)

        def row(r, _):
            for c in range(D // 16):
                a = acc[blk * WB + r, pl.ds(c * 16, 16)]
                az = jnp.where(a == NEG_INF, 0.0, a)
                s = wbuf[r, pl.ds(c * 16, 16)] + az
                wbuf[r, pl.ds(c * 16, 16)] = jnp.maximum(s, 0.01 * s)
            return 0
        lax.fori_loop(0, WB, row, 0)
        pltpu.sync_copy(wbuf, out_hbm.at[pl.ds(lo + blk * WB, WB)])
        return 0
    lax.fori_loop(0, NPW // WB, wb, 0)


def _edge_layer(xwu, xw, src, dst):
    mesh = plsc.VectorSubcoreMesh(core_axis_name="c", subcore_axis_name="s")
    f = pl.kernel(
        _edge_body,
        out_type=jax.ShapeDtypeStruct((NP, D), jnp.float32),
        mesh=mesh,
        compiler_params=pltpu.CompilerParams(needs_layout_passes=False, internal_scratch_in_bytes=0),
        scratch_types=[
            pltpu.VMEM((NPW, D), jnp.float32),
            pltpu.VMEM((C,), jnp.int32),
            pltpu.VMEM((C,), jnp.int32),
            pltpu.VMEM((C + B,), jnp.int32),
            pltpu.VMEM((C + B,), jnp.int32),
            pltpu.VMEM((2, B, D), jnp.float32),
            pltpu.VMEM((NPW, D), jnp.float32),
            pltpu.VMEM((NPW,), jnp.int32),
            pltpu.SemaphoreType.DMA,
            pltpu.SemaphoreType.DMA,
        ],
    )
    return f(xwu, xw, src, dst)


def kernel(x, edge_index, edge_attr, Wn1, We1, U1, Wn2, We2, U2):
    ei = edge_index.astype(jnp.int32)
    src = ei[0]
    dst = ei[1]
    xp = jnp.pad(x, ((0, NP - N), (0, 0)))
    xw1, xwu1 = _matmuls(xp, Wn1, U1)
    c1 = _edge_layer(xwu1, xw1, src, dst)
    xw2, xwu2 = _matmuls(c1, Wn2, U2)
    c2 = _edge_layer(xwu2, xw2, src, dst)
    return c2[:N]
